# Initial kernel scaffold; baseline (speedup 1.0000x reference)
import functools

import jax
import jax.numpy as jnp
from jax import lax
from jax.experimental import pallas as pl
from jax.experimental.pallas import tpu as pltpu
from jax.experimental.pallas import tpu_sc as plsc

N = 16384
K = 8192
D = 256
BR = 256
BC = 1024
NCH = K // BC
BETA = 0.25


def _argmin_body(xf_ref, xb_ref, wf_ref, wb_ref, idx_ref, loss_ref, wsq_ref):
    r = pl.program_id(0)

    @pl.when(r == 0)
    def _():
        w = wf_ref[...]
        wsq_ref[...] = jnp.sum(w * w, axis=1)

    x = xf_ref[...]
    xsq = jnp.sum(x * x, axis=1)
    xb = xb_ref[...]

    rmin = jnp.full((BR,), jnp.inf, jnp.float32)
    ridx = jnp.zeros((BR,), jnp.int32)
    for c in range(NCH):
        wchunk = wb_ref[c * BC:(c + 1) * BC, :]
        d = jnp.dot(xb, wchunk.T, preferred_element_type=jnp.float32)
        wsq_c = wsq_ref[c * BC:(c + 1) * BC]
        dist = (xsq[:, None] + wsq_c[None, :]) - 2.0 * d
        lmin = jnp.min(dist, axis=1)
        lane = lax.broadcasted_iota(jnp.int32, (BR, BC), 1)
        lidx = jnp.min(jnp.where(dist == lmin[:, None], lane, K), axis=1) + c * BC
        better = lmin < rmin
        rmin = jnp.where(better, lmin, rmin)
        ridx = jnp.where(better, lidx, ridx)

    idx_ref[...] = ridx

    @pl.when(r == 0)
    def _():
        loss_ref[0, 0] = 0.0

    loss_ref[0, 0] += jnp.sum(rmin)


_argmin_call = pl.pallas_call(
    _argmin_body,
    grid=(N // BR,),
    in_specs=[
        pl.BlockSpec((BR, D), lambda r: (r, 0)),
        pl.BlockSpec((BR, D), lambda r: (r, 0)),
        pl.BlockSpec((K, D), lambda r: (0, 0)),
        pl.BlockSpec((K, D), lambda r: (0, 0)),
    ],
    out_specs=[
        pl.BlockSpec((BR,), lambda r: (r,)),
        pl.BlockSpec((1, 1), lambda r: (0, 0), memory_space=pltpu.SMEM),
    ],
    out_shape=[
        jax.ShapeDtypeStruct((N,), jnp.int32),
        jax.ShapeDtypeStruct((1, 1), jnp.float32),
    ],
    scratch_shapes=[pltpu.VMEM((K,), jnp.float32)],
)


_info = plsc.get_sparse_core_info()
_NC, _NS = _info.num_cores, _info.num_subcores
_NW = _NC * _NS
_BPW = N // _NW
_CH = 128


@functools.partial(
    pl.kernel,
    mesh=plsc.VectorSubcoreMesh(core_axis_name="c", subcore_axis_name="s"),
    out_type=jax.ShapeDtypeStruct((N, D), jnp.float32),
    scratch_types=[
        pltpu.VMEM((_CH,), jnp.int32),
        pltpu.VMEM((_CH, D), jnp.float32),
        pltpu.SemaphoreType.DMA,
    ],
)
def _gather(idx_hbm, table_hbm, out_hbm, idx_v, rows_v, sem):
    wid = lax.axis_index("s") * _NC + lax.axis_index("c")
    base = wid * _BPW
    for j in range(_BPW // _CH):
        off = base + j * _CH
        pltpu.sync_copy(idx_hbm.at[pl.ds(off, _CH)], idx_v)
        pltpu.async_copy(table_hbm.at[idx_v], rows_v, sem).wait()
        pltpu.sync_copy(rows_v, out_hbm.at[pl.ds(off, _CH)])


def kernel(x, weight):
    xt = jnp.transpose(x, (0, 2, 3, 1))
    x_flat = xt.reshape(N, D)
    x_bf = x_flat.astype(jnp.bfloat16)
    w_bf = weight.astype(jnp.bfloat16)

    idx, loss_sum = _argmin_call(x_flat, x_bf, weight, w_bf)

    x_q = _gather(idx, weight)

    out = jnp.transpose(x_q.reshape(16, 32, 32, D), (0, 3, 1, 2))
    loss = (1.0 + BETA) * loss_sum[0, 0] / (N * D)
    return out, loss, idx


# speedup vs baseline: 1.0520x; 1.0520x over previous
import functools

import jax
import jax.numpy as jnp
from jax import lax
from jax.experimental import pallas as pl
from jax.experimental.pallas import tpu as pltpu
from jax.experimental.pallas import tpu_sc as plsc

N = 16384
K = 8192
D = 256
BR = 256
BC = 1024
NCH = K // BC
BETA = 0.25


_WIN = (2736, 5472, 8192)


def _win_of(k):
    for w, hi in enumerate(_WIN):
        if k < hi:
            return w
    return len(_WIN) - 1


def _argmin_body(xf_ref, xb_ref, wf_ref, wb_ref, idx_ref, loss_ref, wsq_ref):
    r = pl.program_id(0)

    @pl.when(r == 0)
    def _():
        w = wf_ref[...]
        wsq_ref[...] = jnp.sum(w * w, axis=1)

    x = xf_ref[...]
    xsq = jnp.sum(x * x, axis=1)
    xb = xb_ref[...]

    inf = jnp.full((BR,), jnp.inf, jnp.float32)
    win_v = [inf, inf, inf]
    win_i = [jnp.zeros((BR,), jnp.int32)] * 3
    for c in range(NCH):
        base = c * BC
        wchunk = wb_ref[base:base + BC, :]
        d = jnp.dot(xb, wchunk.T, preferred_element_type=jnp.float32)
        wsq_c = wsq_ref[base:base + BC]
        dist = (xsq[:, None] + wsq_c[None, :]) - 2.0 * d
        lane = lax.broadcasted_iota(jnp.int32, (BR, BC), 1)
        cuts = sorted({0, BC} | {hi - base for hi in _WIN if 0 < hi - base < BC})
        for s, e in zip(cuts[:-1], cuts[1:]):
            w = _win_of(base + s)
            if s == 0 and e == BC:
                seg = dist
                segsel = dist == jnp.min(dist, axis=1)[:, None]
            else:
                m = (lane >= s) & (lane < e)
                seg = jnp.where(m, dist, jnp.inf)
                segsel = m & (seg == jnp.min(seg, axis=1)[:, None])
            lmin = jnp.min(seg, axis=1)
            lidx = jnp.min(jnp.where(segsel, lane, K), axis=1) + base
            better = lmin < win_v[w]
            win_v[w] = jnp.where(better, lmin, win_v[w])
            win_i[w] = jnp.where(better, lidx, win_i[w])

    def _bf16(v):
        return v.astype(jnp.bfloat16).astype(jnp.float32)

    acc_v = _bf16(win_v[0])
    acc_i = win_i[0]
    acc_t = win_v[0]
    for w in (1, 2):
        take = win_v[w] < acc_v
        acc_i = jnp.where(take, win_i[w], acc_i)
        acc_t = jnp.where(take, win_v[w], acc_t)
        acc_v = _bf16(jnp.where(take, win_v[w], acc_v))

    rmin = acc_t
    idx_ref[...] = acc_i

    @pl.when(r == 0)
    def _():
        loss_ref[0, 0] = 0.0

    loss_ref[0, 0] += jnp.sum(rmin)


_argmin_call = pl.pallas_call(
    _argmin_body,
    grid=(N // BR,),
    in_specs=[
        pl.BlockSpec((BR, D), lambda r: (r, 0)),
        pl.BlockSpec((BR, D), lambda r: (r, 0)),
        pl.BlockSpec((K, D), lambda r: (0, 0)),
        pl.BlockSpec((K, D), lambda r: (0, 0)),
    ],
    out_specs=[
        pl.BlockSpec((BR,), lambda r: (r,)),
        pl.BlockSpec((1, 1), lambda r: (0, 0), memory_space=pltpu.SMEM),
    ],
    out_shape=[
        jax.ShapeDtypeStruct((N,), jnp.int32),
        jax.ShapeDtypeStruct((1, 1), jnp.float32),
    ],
    scratch_shapes=[pltpu.VMEM((K,), jnp.float32)],
)


_NC, _NS = 2, 16
_NW = _NC * _NS
_BPW = N // _NW
_CH = 128


@functools.cache
def _make_gather():
    @functools.partial(
        pl.kernel,
        mesh=plsc.VectorSubcoreMesh(core_axis_name="c", subcore_axis_name="s"),
        out_type=jax.ShapeDtypeStruct((N, D), jnp.float32),
        scratch_types=[
            pltpu.VMEM((_CH,), jnp.int32),
            pltpu.VMEM((_CH, D), jnp.float32),
            pltpu.SemaphoreType.DMA,
        ],
    )
    def _gather(idx_hbm, table_hbm, out_hbm, idx_v, rows_v, sem):
        wid = lax.axis_index("s") * _NC + lax.axis_index("c")
        base = wid * _BPW
        for j in range(_BPW // _CH):
            off = base + j * _CH
            pltpu.sync_copy(idx_hbm.at[pl.ds(off, _CH)], idx_v)
            pltpu.async_copy(table_hbm.at[idx_v], rows_v, sem).wait()
            pltpu.sync_copy(rows_v, out_hbm.at[pl.ds(off, _CH)])

    return _gather


def kernel(x, weight):
    xt = jnp.transpose(x, (0, 2, 3, 1))
    x_flat = xt.reshape(N, D)
    x_bf = x_flat.astype(jnp.bfloat16)
    w_bf = weight.astype(jnp.bfloat16)

    idx, loss_sum = _argmin_call(x_flat, x_bf, weight, w_bf)

    x_q = _make_gather()(idx, weight)

    out = jnp.transpose(x_q.reshape(16, 32, 32, D), (0, 3, 1, 2))
    loss = (1.0 + BETA) * loss_sum[0, 0] / (N * D)
    return out, loss, idx


# speedup vs baseline: 1.1310x; 1.0751x over previous
import functools

import jax
import jax.numpy as jnp
from jax import lax
from jax.experimental import pallas as pl
from jax.experimental.pallas import tpu as pltpu
from jax.experimental.pallas import tpu_sc as plsc

N = 16384
K = 8192
D = 256
BR = 512
BC = 1024
NCH = K // BC
BETA = 0.25


_WIN = (2736, 5472, 8192)


def _win_of(k):
    for w, hi in enumerate(_WIN):
        if k < hi:
            return w
    return len(_WIN) - 1


def _argmin_body(xf_ref, xb_ref, wf_ref, wb_ref, idx_ref, loss_ref, wsq_ref):
    r = pl.program_id(0)

    @pl.when(r == 0)
    def _():
        w = wf_ref[...]
        wsq_ref[...] = jnp.sum(w * w, axis=1)

    x = xf_ref[...]
    xsq = jnp.sum(x * x, axis=1)
    xb = xb_ref[...]

    inf = jnp.full((BR,), jnp.inf, jnp.float32)
    win_v = [inf, inf, inf]
    win_i = [jnp.zeros((BR,), jnp.int32)] * 3
    for c in range(NCH):
        base = c * BC
        wchunk = wb_ref[base:base + BC, :]
        d2 = jnp.dot(xb, wchunk.T, preferred_element_type=jnp.float32)
        wsq_c = wsq_ref[base:base + BC]
        dist = (xsq[:, None] + wsq_c[None, :]) + d2
        lane = lax.broadcasted_iota(jnp.int32, (BR, BC), 1)
        cuts = sorted({0, BC} | {hi - base for hi in _WIN if 0 < hi - base < BC})
        for s, e in zip(cuts[:-1], cuts[1:]):
            w = _win_of(base + s)
            if s == 0 and e == BC:
                seg = dist
                segsel = dist == jnp.min(dist, axis=1)[:, None]
            else:
                m = (lane >= s) & (lane < e)
                seg = jnp.where(m, dist, jnp.inf)
                segsel = m & (seg == jnp.min(seg, axis=1)[:, None])
            lmin = jnp.min(seg, axis=1)
            lidx = jnp.min(jnp.where(segsel, lane, K), axis=1) + base
            better = lmin < win_v[w]
            win_v[w] = jnp.where(better, lmin, win_v[w])
            win_i[w] = jnp.where(better, lidx, win_i[w])

    def _bf16(v):
        return v.astype(jnp.bfloat16).astype(jnp.float32)

    acc_v = _bf16(win_v[0])
    acc_i = win_i[0]
    acc_t = win_v[0]
    for w in (1, 2):
        take = win_v[w] < acc_v
        acc_i = jnp.where(take, win_i[w], acc_i)
        acc_t = jnp.where(take, win_v[w], acc_t)
        acc_v = _bf16(jnp.where(take, win_v[w], acc_v))

    rmin = acc_t
    idx_ref[...] = acc_i

    @pl.when(r == 0)
    def _():
        loss_ref[0, 0] = 0.0

    loss_ref[0, 0] += jnp.sum(rmin)


_argmin_call = pl.pallas_call(
    _argmin_body,
    grid=(N // BR,),
    in_specs=[
        pl.BlockSpec((BR, D), lambda r: (r, 0)),
        pl.BlockSpec((BR, D), lambda r: (r, 0)),
        pl.BlockSpec((K, D), lambda r: (0, 0)),
        pl.BlockSpec((K, D), lambda r: (0, 0)),
    ],
    out_specs=[
        pl.BlockSpec((BR,), lambda r: (r,)),
        pl.BlockSpec((1, 1), lambda r: (0, 0), memory_space=pltpu.SMEM),
    ],
    out_shape=[
        jax.ShapeDtypeStruct((N,), jnp.int32),
        jax.ShapeDtypeStruct((1, 1), jnp.float32),
    ],
    scratch_shapes=[pltpu.VMEM((K,), jnp.float32)],
)


_NC, _NS = 2, 16
_NW = _NC * _NS
_BPW = N // _NW
_CH = 128


@functools.cache
def _make_gather():
    @functools.partial(
        pl.kernel,
        mesh=plsc.VectorSubcoreMesh(core_axis_name="c", subcore_axis_name="s"),
        out_type=jax.ShapeDtypeStruct((N, D), jnp.float32),
        scratch_types=[
            pltpu.VMEM((_CH,), jnp.int32),
            pltpu.VMEM((_CH, D), jnp.float32),
            pltpu.SemaphoreType.DMA,
        ],
    )
    def _gather(idx_hbm, table_hbm, out_hbm, idx_v, rows_v, sem):
        wid = lax.axis_index("s") * _NC + lax.axis_index("c")
        base = wid * _BPW
        for j in range(_BPW // _CH):
            off = base + j * _CH
            pltpu.sync_copy(idx_hbm.at[pl.ds(off, _CH)], idx_v)
            pltpu.async_copy(table_hbm.at[idx_v], rows_v, sem).wait()
            pltpu.sync_copy(rows_v, out_hbm.at[pl.ds(off, _CH)])

    return _gather


def kernel(x, weight):
    xt = jnp.transpose(x, (0, 2, 3, 1))
    x_flat = xt.reshape(N, D)
    x_bf = x_flat.astype(jnp.bfloat16)
    w_bf = (weight * -2.0).astype(jnp.bfloat16)

    idx, loss_sum = _argmin_call(x_flat, x_bf, weight, w_bf)

    x_q = _make_gather()(idx, weight)

    out = jnp.transpose(x_q.reshape(16, 32, 32, D), (0, 3, 1, 2))
    loss = (1.0 + BETA) * loss_sum[0, 0] / (N * D)
    return out, loss, idx


# speedup vs baseline: 1.1741x; 1.0381x over previous
import functools

import jax
import jax.numpy as jnp
from jax import lax
from jax.experimental import pallas as pl
from jax.experimental.pallas import tpu as pltpu
from jax.experimental.pallas import tpu_sc as plsc

N = 16384
K = 8192
D = 256
BR = 512
BC = 1024
NCH = K // BC
BETA = 0.25


_WIN = (2736, 5472, 8192)


def _win_of(k):
    for w, hi in enumerate(_WIN):
        if k < hi:
            return w
    return len(_WIN) - 1


def _argmin_body(xf_ref, xb_ref, wf_ref, wb_ref, idx_ref, loss_ref, wsq_ref):
    r = pl.program_id(0)

    @pl.when(r == 0)
    def _():
        w = wf_ref[...]
        wsq_ref[...] = jnp.sum(w * w, axis=1)

    x = xf_ref[...]
    xsq = jnp.sum(x * x, axis=1, keepdims=True)
    xb = xb_ref[...]

    def chunk_dist(c):
        base = c * BC
        wchunk = wb_ref[:, base:base + BC]
        d2 = jnp.dot(xb, wchunk, preferred_element_type=jnp.float32)
        wsq_c = wsq_ref[base:base + BC]
        return (xsq + wsq_c[None, :]) + d2

    def chunk_cuts(c):
        base = c * BC
        cuts = sorted({0, BC} | {hi - base for hi in _WIN if 0 < hi - base < BC})
        return [(s, e, _win_of(base + s)) for s, e in zip(cuts[:-1], cuts[1:])]

    lane = lax.broadcasted_iota(jnp.int32, (BR, BC), 1).astype(jnp.float32)
    inf = jnp.full((BR, 1), jnp.inf, jnp.float32)

    win_v = [inf, inf, inf]
    for c in range(NCH):
        dist = chunk_dist(c)
        for s, e, w in chunk_cuts(c):
            if s == 0 and e == BC:
                seg = dist
            else:
                seg = jnp.where((lane >= s) & (lane < e), dist, jnp.inf)
            win_v[w] = jnp.minimum(win_v[w], jnp.min(seg, axis=1, keepdims=True))

    win_i = [jnp.full((BR, 1), K, jnp.int32)] * 3
    for c in range(NCH):
        dist = chunk_dist(c)
        base = c * BC
        for s, e, w in chunk_cuts(c):
            hit = dist == win_v[w]
            if not (s == 0 and e == BC):
                hit &= (lane >= float(s)) & (lane < float(e))
            lidx = jnp.min(jnp.where(hit, lane, float(BC)),
                           axis=1, keepdims=True).astype(jnp.int32)
            win_i[w] = jnp.minimum(
                win_i[w], jnp.where(lidx < BC, lidx + base, K))

    def _bf16(v):
        return v.astype(jnp.bfloat16).astype(jnp.float32)

    acc_v = _bf16(win_v[0])
    acc_i = win_i[0]
    acc_t = win_v[0]
    for w in (1, 2):
        take = win_v[w] < acc_v
        acc_i = jnp.where(take, win_i[w], acc_i)
        acc_t = jnp.where(take, win_v[w], acc_t)
        acc_v = _bf16(jnp.where(take, win_v[w], acc_v))

    rmin = acc_t
    idx_ref[...] = acc_i[:, 0]

    @pl.when(r == 0)
    def _():
        loss_ref[0, 0] = 0.0

    loss_ref[0, 0] += jnp.sum(rmin)


_argmin_call = pl.pallas_call(
    _argmin_body,
    grid=(N // BR,),
    in_specs=[
        pl.BlockSpec((BR, D), lambda r: (r, 0)),
        pl.BlockSpec((BR, D), lambda r: (r, 0)),
        pl.BlockSpec((K, D), lambda r: (0, 0)),
        pl.BlockSpec((D, K), lambda r: (0, 0)),
    ],
    out_specs=[
        pl.BlockSpec((BR,), lambda r: (r,)),
        pl.BlockSpec((1, 1), lambda r: (0, 0), memory_space=pltpu.SMEM),
    ],
    out_shape=[
        jax.ShapeDtypeStruct((N,), jnp.int32),
        jax.ShapeDtypeStruct((1, 1), jnp.float32),
    ],
    scratch_shapes=[pltpu.VMEM((K,), jnp.float32)],
)


_NC, _NS = 2, 16
_NW = _NC * _NS
_BPW = N // _NW
_CH = 128


@functools.cache
def _make_gather():
    @functools.partial(
        pl.kernel,
        mesh=plsc.VectorSubcoreMesh(core_axis_name="c", subcore_axis_name="s"),
        out_type=jax.ShapeDtypeStruct((N, D), jnp.float32),
        scratch_types=[
            pltpu.VMEM((_CH,), jnp.int32),
            pltpu.VMEM((_CH, D), jnp.float32),
            pltpu.SemaphoreType.DMA,
        ],
    )
    def _gather(idx_hbm, table_hbm, out_hbm, idx_v, rows_v, sem):
        wid = lax.axis_index("s") * _NC + lax.axis_index("c")
        base = wid * _BPW
        for j in range(_BPW // _CH):
            off = base + j * _CH
            pltpu.sync_copy(idx_hbm.at[pl.ds(off, _CH)], idx_v)
            pltpu.async_copy(table_hbm.at[idx_v], rows_v, sem).wait()
            pltpu.sync_copy(rows_v, out_hbm.at[pl.ds(off, _CH)])

    return _gather


def kernel(x, weight):
    xt = jnp.transpose(x, (0, 2, 3, 1))
    x_flat = xt.reshape(N, D)
    x_bf = x_flat.astype(jnp.bfloat16)
    w_bf = (weight * -2.0).astype(jnp.bfloat16).T

    idx, loss_sum = _argmin_call(x_flat, x_bf, weight, w_bf)

    x_q = _make_gather()(idx, weight)

    out = jnp.transpose(x_q.reshape(16, 32, 32, D), (0, 3, 1, 2))
    loss = (1.0 + BETA) * loss_sum[0, 0] / (N * D)
    return out, loss, idx


# speedup vs baseline: 1.2875x; 1.0966x over previous
import functools

import jax
import jax.numpy as jnp
from jax import lax
from jax.experimental import pallas as pl
from jax.experimental.pallas import tpu as pltpu
from jax.experimental.pallas import tpu_sc as plsc

N = 16384
K = 8192
D = 256
BR = 512
KP = 2816
KT = 3 * KP
BETA = 0.25


_WIN = (2736, 5472, 8192)
_WBASE = (0, 2736, 5472)
_WLEN = (2736, 2736, 2720)


def _argmin_body(xf_ref, xb_ref, wf_ref, wb_ref, idx_ref, loss_ref, wsq_ref):
    r = pl.program_id(0)

    @pl.when(r == 0)
    def _():
        w = wf_ref[...]
        wsq_ref[...] = jnp.sum(w * w, axis=1)

    x = xf_ref[...]
    xsq = jnp.sum(x * x, axis=1, keepdims=True)
    xb = xb_ref[...]

    def region_dist(g):
        base = g * KP
        wchunk = wb_ref[:, base:base + KP]
        d2 = jnp.dot(xb, wchunk, preferred_element_type=jnp.float32)
        wsq_c = wsq_ref[base:base + KP]
        return (xsq + wsq_c[None, :]) + d2

    lane = lax.broadcasted_iota(jnp.int32, (BR, KP), 1).astype(jnp.float32)

    win_v = [jnp.min(region_dist(g), axis=1, keepdims=True) for g in range(3)]

    win_i = []
    for g in range(3):
        hit = region_dist(g) == win_v[g]
        lidx = jnp.min(jnp.where(hit, lane, float(KP)),
                       axis=1, keepdims=True).astype(jnp.int32)
        win_i.append(lidx + _WBASE[g])

    def _bf16(v):
        return v.astype(jnp.bfloat16).astype(jnp.float32)

    acc_v = _bf16(win_v[0])
    acc_i = win_i[0]
    acc_t = win_v[0]
    for w in (1, 2):
        take = win_v[w] < acc_v
        acc_i = jnp.where(take, win_i[w], acc_i)
        acc_t = jnp.where(take, win_v[w], acc_t)
        acc_v = _bf16(jnp.where(take, win_v[w], acc_v))

    rmin = acc_t
    idx_ref[...] = acc_i[:, 0]

    @pl.when(r == 0)
    def _():
        loss_ref[0, 0] = 0.0

    loss_ref[0, 0] += jnp.sum(rmin)


_argmin_call = pl.pallas_call(
    _argmin_body,
    grid=(N // BR,),
    in_specs=[
        pl.BlockSpec((BR, D), lambda r: (r, 0)),
        pl.BlockSpec((BR, D), lambda r: (r, 0)),
        pl.BlockSpec((KT, D), lambda r: (0, 0)),
        pl.BlockSpec((D, KT), lambda r: (0, 0)),
    ],
    out_specs=[
        pl.BlockSpec((BR,), lambda r: (r,)),
        pl.BlockSpec((1, 1), lambda r: (0, 0), memory_space=pltpu.SMEM),
    ],
    out_shape=[
        jax.ShapeDtypeStruct((N,), jnp.int32),
        jax.ShapeDtypeStruct((1, 1), jnp.float32),
    ],
    scratch_shapes=[pltpu.VMEM((KT,), jnp.float32)],
)


_NC, _NS = 2, 16
_NW = _NC * _NS
_BPW = N // _NW
_CH = 128


@functools.cache
def _make_gather():
    @functools.partial(
        pl.kernel,
        mesh=plsc.VectorSubcoreMesh(core_axis_name="c", subcore_axis_name="s"),
        out_type=jax.ShapeDtypeStruct((N, D), jnp.float32),
        scratch_types=[
            pltpu.VMEM((_CH,), jnp.int32),
            pltpu.VMEM((_CH, D), jnp.float32),
            pltpu.SemaphoreType.DMA,
        ],
    )
    def _gather(idx_hbm, table_hbm, out_hbm, idx_v, rows_v, sem):
        wid = lax.axis_index("s") * _NC + lax.axis_index("c")
        base = wid * _BPW
        for j in range(_BPW // _CH):
            off = base + j * _CH
            pltpu.sync_copy(idx_hbm.at[pl.ds(off, _CH)], idx_v)
            pltpu.async_copy(table_hbm.at[idx_v], rows_v, sem).wait()
            pltpu.sync_copy(rows_v, out_hbm.at[pl.ds(off, _CH)])

    return _gather


def kernel(x, weight):
    xt = jnp.transpose(x, (0, 2, 3, 1))
    x_flat = xt.reshape(N, D)
    x_bf = x_flat.astype(jnp.bfloat16)
    wbt = (weight * -2.0).astype(jnp.bfloat16).T

    zb = jnp.zeros((D, KP), jnp.bfloat16)
    w_pad_t = jnp.concatenate(
        [jnp.concatenate([wbt[:, _WBASE[g]:_WBASE[g] + _WLEN[g]],
                          zb[:, :KP - _WLEN[g]]], axis=1)
         for g in range(3)], axis=1)
    zf = jnp.full((KP, D), jnp.inf, jnp.float32)
    w_pad_f = jnp.concatenate(
        [jnp.concatenate([weight[_WBASE[g]:_WBASE[g] + _WLEN[g]],
                          zf[:KP - _WLEN[g]]], axis=0)
         for g in range(3)], axis=0)

    idx, loss_sum = _argmin_call(x_flat, x_bf, w_pad_f, w_pad_t)

    x_q = _make_gather()(idx, weight)

    out = jnp.transpose(x_q.reshape(16, 32, 32, D), (0, 3, 1, 2))
    loss = (1.0 + BETA) * loss_sum[0, 0] / (N * D)
    return out, loss, idx
